# baseline (speedup 1.0000x reference)
import jax
import jax.numpy as jnp
from jax.experimental import pallas as pl

_G = 4
_NP = 32
_BD = _G * _NP
_K = 10
_N = 30


def _body(real_ref, graph_ref, w1t_ref, b1_ref, w2t_ref, b2_ref,
          wlin_ref, blin_ref, wconv_ref, bconv_ref, out_ref):
    f32 = jnp.float32
    gr = graph_ref[...]
    adj = jnp.mean(gr, axis=1)
    ir = jax.lax.broadcasted_iota(jnp.int32, (_N, _N), 0)
    ic = jax.lax.broadcasted_iota(jnp.int32, (_N, _N), 1)
    eye30 = jnp.where(ir == ic, 1.0, 0.0).astype(f32)
    a_hat = (adj != 0).astype(f32) + eye30[None]
    deg = jnp.sum(a_hat, axis=2)
    dinv = jax.lax.rsqrt(deg)
    norm = dinv[:, :, None] * a_hat * dinv[:, None, :]
    normp = jnp.pad(norm, ((0, 0), (0, 2), (0, 2)))
    flat = normp.reshape(_BD, _NP)
    tiled = jnp.concatenate([flat] * _G, axis=1)
    row = jax.lax.broadcasted_iota(jnp.int32, (_BD, _BD), 0)
    col = jax.lax.broadcasted_iota(jnp.int32, (_BD, _BD), 1)
    bd = jnp.where((row // _NP) == (col // _NP), tiled, 0.0)
    a_t = bd.T
    ipad = jnp.where((row == col) & (row % _NP < _N), 1.0, 0.0).astype(f32)

    xp = jnp.pad(real_ref[...], ((0, 0), (0, 2), (0, 0))).reshape(_BD, 128)
    h1 = jnp.maximum(
        jnp.dot(xp, w1t_ref[...], preferred_element_type=f32) + b1_ref[...], 0.0)
    z = jnp.dot(h1, w2t_ref[...], preferred_element_type=f32)

    def step(_, t):
        return 0.9 * jnp.dot(a_t, t, preferred_element_type=f32) + 0.1 * ipad

    t10 = jax.lax.fori_loop(0, _K, step, ipad)

    y = jnp.maximum(
        jnp.dot(t10, z, preferred_element_type=f32) + b2_ref[...], 0.0)
    v = jnp.dot(y, wlin_ref[...], preferred_element_type=f32)
    u = jnp.dot(t10, v, preferred_element_type=f32)
    xf = jnp.maximum(u + blin_ref[0, 0], 0.0)
    prod = xf * wconv_ref[...]
    out_ref[0] = jnp.sum(prod.reshape(_G, _NP, 4), axis=1) + bconv_ref[...]


def kernel(real, imag, graph, layer, W1, b1, W2, b2, Wlin, blin, Wconv, bconv):
    del imag, layer
    B = real.shape[0]
    w1t = W1.T
    w2t = W2.T
    b1r = b1.reshape(1, 128)
    b2r = b2.reshape(1, 128)
    wlin_c = Wlin.reshape(128, 1)
    blin_r = blin.reshape(1, 1)
    wconv_pad = jnp.pad(Wconv[:, :, 0].T, ((0, 2), (0, 0)))
    wconv_big = jnp.tile(wconv_pad, (_G, 1))
    bconv_r = bconv.reshape(1, 4)
    grid = (B // _G,)
    return pl.pallas_call(
        _body,
        grid=grid,
        in_specs=[
            pl.BlockSpec((_G, _N, 128), lambda i: (i, 0, 0)),
            pl.BlockSpec((_G, 4, _N, _N), lambda i: (i, 0, 0, 0)),
            pl.BlockSpec((128, 128), lambda i: (0, 0)),
            pl.BlockSpec((1, 128), lambda i: (0, 0)),
            pl.BlockSpec((128, 128), lambda i: (0, 0)),
            pl.BlockSpec((1, 128), lambda i: (0, 0)),
            pl.BlockSpec((128, 1), lambda i: (0, 0)),
            pl.BlockSpec((1, 1), lambda i: (0, 0)),
            pl.BlockSpec((128, 4), lambda i: (0, 0)),
            pl.BlockSpec((1, 4), lambda i: (0, 0)),
        ],
        out_specs=pl.BlockSpec((1, _G, 4), lambda i: (i, 0, 0)),
        out_shape=jax.ShapeDtypeStruct((B // _G, _G, 4), jnp.float32),
    )(real, graph, w1t, b1r, w2t, b2r, wlin_c, blin_r, wconv_big, bconv_r).reshape(B, 4)


# speedup vs baseline: 1.3611x; 1.3611x over previous
import jax
import jax.numpy as jnp
from jax.experimental import pallas as pl

_G = 8
_NP = 32
_BD = _G * _NP
_K = 10
_N = 30


def _body(real_ref, graph_ref, w1t_ref, b1_ref, w2t_ref, b2_ref,
          wlin_ref, blin_ref, wconv_ref, bconv_ref, out_ref):
    f32 = jnp.float32
    gr = graph_ref[...]
    adj = jnp.mean(gr, axis=1)
    ir = jax.lax.broadcasted_iota(jnp.int32, (_N, _N), 0)
    ic = jax.lax.broadcasted_iota(jnp.int32, (_N, _N), 1)
    eye30 = jnp.where(ir == ic, 1.0, 0.0).astype(f32)
    a_hat = (adj != 0).astype(f32) + eye30[None]
    deg = jnp.sum(a_hat, axis=2)
    dinv = jax.lax.rsqrt(deg)
    norm = dinv[:, :, None] * a_hat * dinv[:, None, :]
    normp = jnp.pad(norm, ((0, 0), (0, 2), (0, 2)))
    flat = normp.reshape(_BD, _NP)
    tiled = jnp.concatenate([flat] * _G, axis=1)
    row = jax.lax.broadcasted_iota(jnp.int32, (_BD, _BD), 0)
    col = jax.lax.broadcasted_iota(jnp.int32, (_BD, _BD), 1)
    bd = jnp.where((row // _NP) == (col // _NP), tiled, 0.0)
    a_t = bd.T
    ipad = jnp.where((row == col) & (row % _NP < _N), 1.0, 0.0).astype(f32)

    xp = jnp.pad(real_ref[...], ((0, 0), (0, 2), (0, 0))).reshape(_BD, 128)
    h1 = jnp.maximum(
        jnp.dot(xp, w1t_ref[...], preferred_element_type=f32) + b1_ref[...], 0.0)
    z = jnp.dot(h1, w2t_ref[...], preferred_element_type=f32)

    def step(_, t):
        return 0.9 * jnp.dot(a_t, t, preferred_element_type=f32) + 0.1 * ipad

    t10 = jax.lax.fori_loop(0, _K, step, ipad)

    y = jnp.maximum(
        jnp.dot(t10, z, preferred_element_type=f32) + b2_ref[...], 0.0)
    v = jnp.dot(y, wlin_ref[...], preferred_element_type=f32)
    u = jnp.dot(t10, v, preferred_element_type=f32)
    xf = jnp.maximum(u + blin_ref[0, 0], 0.0)
    prod = xf * wconv_ref[...]
    out_ref[0] = jnp.sum(prod.reshape(_G, _NP, 4), axis=1) + bconv_ref[...]


def kernel(real, imag, graph, layer, W1, b1, W2, b2, Wlin, blin, Wconv, bconv):
    del imag, layer
    B = real.shape[0]
    w1t = W1.T
    w2t = W2.T
    b1r = b1.reshape(1, 128)
    b2r = b2.reshape(1, 128)
    wlin_c = Wlin.reshape(128, 1)
    blin_r = blin.reshape(1, 1)
    wconv_pad = jnp.pad(Wconv[:, :, 0].T, ((0, 2), (0, 0)))
    wconv_big = jnp.tile(wconv_pad, (_G, 1))
    bconv_r = bconv.reshape(1, 4)
    grid = (B // _G,)
    return pl.pallas_call(
        _body,
        grid=grid,
        in_specs=[
            pl.BlockSpec((_G, _N, 128), lambda i: (i, 0, 0)),
            pl.BlockSpec((_G, 4, _N, _N), lambda i: (i, 0, 0, 0)),
            pl.BlockSpec((128, 128), lambda i: (0, 0)),
            pl.BlockSpec((1, 128), lambda i: (0, 0)),
            pl.BlockSpec((128, 128), lambda i: (0, 0)),
            pl.BlockSpec((1, 128), lambda i: (0, 0)),
            pl.BlockSpec((128, 1), lambda i: (0, 0)),
            pl.BlockSpec((1, 1), lambda i: (0, 0)),
            pl.BlockSpec((_BD, 4), lambda i: (0, 0)),
            pl.BlockSpec((1, 4), lambda i: (0, 0)),
        ],
        out_specs=pl.BlockSpec((1, _G, 4), lambda i: (i, 0, 0)),
        out_shape=jax.ShapeDtypeStruct((B // _G, _G, 4), jnp.float32),
    )(real, graph, w1t, b1r, w2t, b2r, wlin_c, blin_r, wconv_big, bconv_r).reshape(B, 4)


# speedup vs baseline: 1.8907x; 1.3892x over previous
import jax
import jax.numpy as jnp
from jax.experimental import pallas as pl

_G = 8
_NP = 32
_BD = _G * _NP
_NG = 2
_GA = _G * _NG
_K = 10
_N = 30


def _body(real_ref, graph_ref, w1t_ref, b1_ref, w2t_ref, b2_ref,
          wlin_ref, blin_ref, wconv_ref, bconv_ref, out_ref):
    f32 = jnp.float32
    ir = jax.lax.broadcasted_iota(jnp.int32, (_N, _N), 0)
    ic = jax.lax.broadcasted_iota(jnp.int32, (_N, _N), 1)
    eye30 = jnp.where(ir == ic, 1.0, 0.0).astype(f32)
    row = jax.lax.broadcasted_iota(jnp.int32, (_BD, _BD), 0)
    col = jax.lax.broadcasted_iota(jnp.int32, (_BD, _BD), 1)
    blkmask = (row // _NP) == (col // _NP)
    ipad = jnp.where((row == col) & (row % _NP < _N), 1.0, 0.0).astype(f32)
    i01 = 0.1 * ipad

    gr = graph_ref[...]
    a9s = []
    for g in range(_NG):
        adj = jnp.mean(gr[g * _G:(g + 1) * _G], axis=1)
        a_hat = (adj != 0).astype(f32) + eye30[None]
        deg = jnp.sum(a_hat, axis=2)
        dinv = jax.lax.rsqrt(deg)
        norm = dinv[:, :, None] * a_hat * dinv[:, None, :]
        normp = jnp.pad(norm, ((0, 0), (0, 2), (0, 2)))
        flat = normp.reshape(_BD, _NP)
        tiled = jnp.concatenate([flat] * _G, axis=1)
        bd = jnp.where(blkmask, tiled, 0.0)
        a9s.append(0.9 * bd.T)

    xp = jnp.pad(real_ref[...], ((0, 0), (0, 2), (0, 0))).reshape(_NG * _BD, 128)
    h1 = jnp.maximum(
        jnp.dot(xp, w1t_ref[...], preferred_element_type=f32) + b1_ref[...], 0.0)
    z = jnp.dot(h1, w2t_ref[...], preferred_element_type=f32)

    def step(_, ts):
        return tuple(
            jnp.dot(a9s[g], ts[g], preferred_element_type=f32) + i01
            for g in range(_NG))

    ts = jax.lax.fori_loop(0, _K, step, (ipad,) * _NG)

    ys = [jnp.maximum(
        jnp.dot(ts[g], z[g * _BD:(g + 1) * _BD], preferred_element_type=f32)
        + b2_ref[...], 0.0) for g in range(_NG)]
    y = jnp.concatenate(ys, axis=0)
    v = jnp.dot(y, wlin_ref[...], preferred_element_type=f32)
    us = [jnp.dot(ts[g], v[g * _BD:(g + 1) * _BD], preferred_element_type=f32)
          for g in range(_NG)]
    u = jnp.concatenate(us, axis=0)
    xf = jnp.maximum(u + blin_ref[0, 0], 0.0)
    prod = xf * wconv_ref[...]
    out_ref[0] = jnp.sum(prod.reshape(_GA, _NP, 4), axis=1) + bconv_ref[...]


def kernel(real, imag, graph, layer, W1, b1, W2, b2, Wlin, blin, Wconv, bconv):
    del imag, layer
    B = real.shape[0]
    w1t = W1.T
    w2t = W2.T
    b1r = b1.reshape(1, 128)
    b2r = b2.reshape(1, 128)
    wlin_c = Wlin.reshape(128, 1)
    blin_r = blin.reshape(1, 1)
    wconv_pad = jnp.pad(Wconv[:, :, 0].T, ((0, 2), (0, 0)))
    wconv_big = jnp.tile(wconv_pad, (_GA, 1))
    bconv_r = bconv.reshape(1, 4)
    grid = (B // _GA,)
    return pl.pallas_call(
        _body,
        grid=grid,
        in_specs=[
            pl.BlockSpec((_GA, _N, 128), lambda i: (i, 0, 0)),
            pl.BlockSpec((_GA, 4, _N, _N), lambda i: (i, 0, 0, 0)),
            pl.BlockSpec((128, 128), lambda i: (0, 0)),
            pl.BlockSpec((1, 128), lambda i: (0, 0)),
            pl.BlockSpec((128, 128), lambda i: (0, 0)),
            pl.BlockSpec((1, 128), lambda i: (0, 0)),
            pl.BlockSpec((128, 1), lambda i: (0, 0)),
            pl.BlockSpec((1, 1), lambda i: (0, 0)),
            pl.BlockSpec((_GA * _NP, 4), lambda i: (0, 0)),
            pl.BlockSpec((1, 4), lambda i: (0, 0)),
        ],
        out_specs=pl.BlockSpec((1, _GA, 4), lambda i: (i, 0, 0)),
        out_shape=jax.ShapeDtypeStruct((B // _GA, _GA, 4), jnp.float32),
    )(real, graph, w1t, b1r, w2t, b2r, wlin_c, blin_r, wconv_big, bconv_r
      ).reshape(B, 4)


# speedup vs baseline: 2.3600x; 1.2482x over previous
import jax
import jax.numpy as jnp
from jax.experimental import pallas as pl

_G = 8
_NP = 32
_BD = _G * _NP
_NG = 4
_GA = _G * _NG
_K = 10
_N = 30


def _body(real_ref, graph_ref, w1t_ref, b1_ref, w2t_ref, b2_ref,
          wlin_ref, blin_ref, wconv_ref, bconv_ref, out_ref):
    f32 = jnp.float32
    ir = jax.lax.broadcasted_iota(jnp.int32, (_N, _N), 0)
    ic = jax.lax.broadcasted_iota(jnp.int32, (_N, _N), 1)
    eye30 = jnp.where(ir == ic, 1.0, 0.0).astype(f32)
    row = jax.lax.broadcasted_iota(jnp.int32, (_BD, _BD), 0)
    col = jax.lax.broadcasted_iota(jnp.int32, (_BD, _BD), 1)
    blkmask = (row // _NP) == (col // _NP)
    ipad = jnp.where((row == col) & (row % _NP < _N), 1.0, 0.0).astype(f32)
    i01 = 0.1 * ipad

    gr = graph_ref[...]
    a9s = []
    for g in range(_NG):
        adj = jnp.mean(gr[g * _G:(g + 1) * _G], axis=1)
        a_hat = (adj != 0).astype(f32) + eye30[None]
        deg = jnp.sum(a_hat, axis=2)
        dinv = jax.lax.rsqrt(deg)
        norm = dinv[:, :, None] * a_hat * dinv[:, None, :]
        normp = jnp.pad(norm, ((0, 0), (0, 2), (0, 2)))
        flat = normp.reshape(_BD, _NP)
        tiled = jnp.concatenate([flat] * _G, axis=1)
        bd = jnp.where(blkmask, tiled, 0.0)
        a9s.append(0.9 * bd.T)

    xp = jnp.pad(real_ref[...], ((0, 0), (0, 2), (0, 0))).reshape(_NG * _BD, 128)
    h1 = jnp.maximum(
        jnp.dot(xp, w1t_ref[...], preferred_element_type=f32) + b1_ref[...], 0.0)
    z = jnp.dot(h1, w2t_ref[...], preferred_element_type=f32)

    def step(_, ts):
        return tuple(
            jnp.dot(a9s[g], ts[g], preferred_element_type=f32) + i01
            for g in range(_NG))

    ts = jax.lax.fori_loop(0, _K, step, (ipad,) * _NG)

    ys = [jnp.maximum(
        jnp.dot(ts[g], z[g * _BD:(g + 1) * _BD], preferred_element_type=f32)
        + b2_ref[...], 0.0) for g in range(_NG)]
    y = jnp.concatenate(ys, axis=0)
    v = jnp.dot(y, wlin_ref[...], preferred_element_type=f32)
    us = [jnp.dot(ts[g], v[g * _BD:(g + 1) * _BD], preferred_element_type=f32)
          for g in range(_NG)]
    u = jnp.concatenate(us, axis=0)
    xf = jnp.maximum(u + blin_ref[0, 0], 0.0)
    prod = xf * wconv_ref[...]
    out_ref[0] = jnp.sum(prod.reshape(_GA, _NP, 4), axis=1) + bconv_ref[...]


def kernel(real, imag, graph, layer, W1, b1, W2, b2, Wlin, blin, Wconv, bconv):
    del imag, layer
    B = real.shape[0]
    w1t = W1.T
    w2t = W2.T
    b1r = b1.reshape(1, 128)
    b2r = b2.reshape(1, 128)
    wlin_c = Wlin.reshape(128, 1)
    blin_r = blin.reshape(1, 1)
    wconv_pad = jnp.pad(Wconv[:, :, 0].T, ((0, 2), (0, 0)))
    wconv_big = jnp.tile(wconv_pad, (_GA, 1))
    bconv_r = bconv.reshape(1, 4)
    grid = (B // _GA,)
    return pl.pallas_call(
        _body,
        grid=grid,
        in_specs=[
            pl.BlockSpec((_GA, _N, 128), lambda i: (i, 0, 0)),
            pl.BlockSpec((_GA, 4, _N, _N), lambda i: (i, 0, 0, 0)),
            pl.BlockSpec((128, 128), lambda i: (0, 0)),
            pl.BlockSpec((1, 128), lambda i: (0, 0)),
            pl.BlockSpec((128, 128), lambda i: (0, 0)),
            pl.BlockSpec((1, 128), lambda i: (0, 0)),
            pl.BlockSpec((128, 1), lambda i: (0, 0)),
            pl.BlockSpec((1, 1), lambda i: (0, 0)),
            pl.BlockSpec((_GA * _NP, 4), lambda i: (0, 0)),
            pl.BlockSpec((1, 4), lambda i: (0, 0)),
        ],
        out_specs=pl.BlockSpec((1, _GA, 4), lambda i: (i, 0, 0)),
        out_shape=jax.ShapeDtypeStruct((B // _GA, _GA, 4), jnp.float32),
    )(real, graph, w1t, b1r, w2t, b2r, wlin_c, blin_r, wconv_big, bconv_r
      ).reshape(B, 4)


# speedup vs baseline: 2.6946x; 1.1418x over previous
import jax
import jax.numpy as jnp
from jax.experimental import pallas as pl

_G = 8
_NP = 32
_BD = _G * _NP
_NG = 8
_GA = _G * _NG
_K = 10
_N = 30


def _body(real_ref, graph_ref, w1t_ref, b1_ref, w2t_ref, b2_ref,
          wlin_ref, blin_ref, wconv_ref, bconv_ref, out_ref):
    f32 = jnp.float32
    ir = jax.lax.broadcasted_iota(jnp.int32, (_N, _N), 0)
    ic = jax.lax.broadcasted_iota(jnp.int32, (_N, _N), 1)
    eye30 = jnp.where(ir == ic, 1.0, 0.0).astype(f32)
    row = jax.lax.broadcasted_iota(jnp.int32, (_BD, _BD), 0)
    col = jax.lax.broadcasted_iota(jnp.int32, (_BD, _BD), 1)
    blkmask = (row // _NP) == (col // _NP)
    ipad = jnp.where((row == col) & (row % _NP < _N), 1.0, 0.0).astype(f32)
    i01 = 0.1 * ipad

    gr = graph_ref[...]
    a9s = []
    for g in range(_NG):
        adj = jnp.mean(gr[g * _G:(g + 1) * _G], axis=1)
        a_hat = (adj != 0).astype(f32) + eye30[None]
        deg = jnp.sum(a_hat, axis=2)
        dinv = jax.lax.rsqrt(deg)
        norm = dinv[:, :, None] * a_hat * dinv[:, None, :]
        normp = jnp.pad(norm, ((0, 0), (0, 2), (0, 2)))
        flat = normp.reshape(_BD, _NP)
        tiled = jnp.concatenate([flat] * _G, axis=1)
        bd = jnp.where(blkmask, tiled, 0.0)
        a9s.append(0.9 * bd.T)

    xp = jnp.pad(real_ref[...], ((0, 0), (0, 2), (0, 0))).reshape(_NG * _BD, 128)
    h1 = jnp.maximum(
        jnp.dot(xp, w1t_ref[...], preferred_element_type=f32) + b1_ref[...], 0.0)
    z = jnp.dot(h1, w2t_ref[...], preferred_element_type=f32)

    def step(_, ts):
        return tuple(
            jnp.dot(a9s[g], ts[g], preferred_element_type=f32) + i01
            for g in range(_NG))

    ts = jax.lax.fori_loop(0, _K, step, (ipad,) * _NG)

    ys = [jnp.maximum(
        jnp.dot(ts[g], z[g * _BD:(g + 1) * _BD], preferred_element_type=f32)
        + b2_ref[...], 0.0) for g in range(_NG)]
    y = jnp.concatenate(ys, axis=0)
    v = jnp.dot(y, wlin_ref[...], preferred_element_type=f32)
    us = [jnp.dot(ts[g], v[g * _BD:(g + 1) * _BD], preferred_element_type=f32)
          for g in range(_NG)]
    u = jnp.concatenate(us, axis=0)
    xf = jnp.maximum(u + blin_ref[0, 0], 0.0)
    prod = xf * wconv_ref[...]
    out_ref[0] = jnp.sum(prod.reshape(_GA, _NP, 4), axis=1) + bconv_ref[...]


def kernel(real, imag, graph, layer, W1, b1, W2, b2, Wlin, blin, Wconv, bconv):
    del imag, layer
    B = real.shape[0]
    w1t = W1.T
    w2t = W2.T
    b1r = b1.reshape(1, 128)
    b2r = b2.reshape(1, 128)
    wlin_c = Wlin.reshape(128, 1)
    blin_r = blin.reshape(1, 1)
    wconv_pad = jnp.pad(Wconv[:, :, 0].T, ((0, 2), (0, 0)))
    wconv_big = jnp.tile(wconv_pad, (_GA, 1))
    bconv_r = bconv.reshape(1, 4)
    grid = (B // _GA,)
    return pl.pallas_call(
        _body,
        grid=grid,
        in_specs=[
            pl.BlockSpec((_GA, _N, 128), lambda i: (i, 0, 0)),
            pl.BlockSpec((_GA, 4, _N, _N), lambda i: (i, 0, 0, 0)),
            pl.BlockSpec((128, 128), lambda i: (0, 0)),
            pl.BlockSpec((1, 128), lambda i: (0, 0)),
            pl.BlockSpec((128, 128), lambda i: (0, 0)),
            pl.BlockSpec((1, 128), lambda i: (0, 0)),
            pl.BlockSpec((128, 1), lambda i: (0, 0)),
            pl.BlockSpec((1, 1), lambda i: (0, 0)),
            pl.BlockSpec((_GA * _NP, 4), lambda i: (0, 0)),
            pl.BlockSpec((1, 4), lambda i: (0, 0)),
        ],
        out_specs=pl.BlockSpec((1, _GA, 4), lambda i: (i, 0, 0)),
        out_shape=jax.ShapeDtypeStruct((B // _GA, _GA, 4), jnp.float32),
    )(real, graph, w1t, b1r, w2t, b2r, wlin_c, blin_r, wconv_big, bconv_r
      ).reshape(B, 4)
